# Initial kernel scaffold; baseline (speedup 1.0000x reference)
import jax
import jax.numpy as jnp
from jax.experimental import pallas as pl


def kernel(x, a, z, ndt, drift_gain, noise):
    raise NotImplementedError("write your pallas kernel here")


# speedup vs baseline: 11.6980x; 11.6980x over previous
import functools
import jax
import jax.numpy as jnp
from jax.experimental import pallas as pl
from jax.experimental.pallas import tpu as pltpu

DT = 0.01
MAX_T = 2.0
STEPS = 200
BN = 2048
BIG = 2**30


def _ddm_block(x_ref, a_ref, z_ref, ndt_ref, g_ref, noise_ref, pr_ref, pc_ref):
    a = a_ref[0, 0]
    z = z_ref[0, 0]
    ndt = ndt_ref[0, 0]
    gain = g_ref[0, 0]

    drift_dt = (gain * DT) * x_ref[...]
    thresh = a - 1e-6

    base = jnp.full((1, BN), z * a, jnp.float32)
    runmin = jnp.full((1, BN), BIG, jnp.int32)

    row_i = jax.lax.broadcasted_iota(jnp.int32, (8, BN), 0)
    row_f = row_i.astype(jnp.float32)

    for c in range(STEPS // 8):
        nz = noise_ref[c * 8:(c + 1) * 8, :]
        s = nz
        for k in (1, 2, 4):
            shifted = jnp.concatenate(
                [jnp.zeros((k, BN), jnp.float32), s[: 8 - k, :]], axis=0)
            s = s + shifted
        tcnt = row_f + jnp.float32(c * 8 + 1)
        dv = base + s + drift_dt * tcnt
        up = dv >= thresh
        crossed = up | (dv <= -thresh)
        enc = jnp.where(
            crossed,
            (row_i + jnp.int32(c * 8)) * 2 + up.astype(jnp.int32),
            jnp.int32(BIG))
        runmin = jnp.minimum(runmin, jnp.min(enc, axis=0, keepdims=True))
        base = base + s[7:8, :]

    hit = runmin < BIG
    t_first = (runmin >> 1).astype(jnp.float32)
    is_up = (runmin & 1).astype(jnp.float32)
    pr_ref[...] = jnp.where(hit, t_first * DT + ndt, MAX_T + ndt)
    pc_ref[...] = jnp.where(hit, is_up, 0.5)


@jax.jit
def kernel(x, a, z, ndt, drift_gain, noise):
    n = x.shape[0]
    x2 = x.reshape(1, n)
    grid = (pl.cdiv(n, BN),)
    scal = pl.BlockSpec(memory_space=pltpu.SMEM)
    pr, pc = pl.pallas_call(
        _ddm_block,
        grid=grid,
        in_specs=[
            pl.BlockSpec((1, BN), lambda i: (0, i)),
            scal, scal, scal, scal,
            pl.BlockSpec((STEPS, BN), lambda i: (0, i)),
        ],
        out_specs=[
            pl.BlockSpec((1, BN), lambda i: (0, i)),
            pl.BlockSpec((1, BN), lambda i: (0, i)),
        ],
        out_shape=[
            jax.ShapeDtypeStruct((1, n), jnp.float32),
            jax.ShapeDtypeStruct((1, n), jnp.float32),
        ],
    )(x2,
      a.reshape(1, 1), z.reshape(1, 1), ndt.reshape(1, 1),
      drift_gain.reshape(1, 1), noise)
    return pr.reshape(n), pc.reshape(n)


# speedup vs baseline: 12.5634x; 1.0740x over previous
import jax
import jax.numpy as jnp
from jax.experimental import pallas as pl
from jax.experimental.pallas import tpu as pltpu

DT = 0.01
MAX_T = 2.0
STEPS = 200
BN = 2048
BIGF = 1e9


def _ddm_block(x_ref, a_ref, z_ref, ndt_ref, g_ref, noise_ref, pr_ref, pc_ref):
    a = a_ref[0, 0]
    z = z_ref[0, 0]
    ndt = ndt_ref[0, 0]
    gain = g_ref[0, 0]

    drift_dt = (gain * DT) * x_ref[...]
    th_hi = (a - 1e-6) - z * a
    th_lo = (-a + 1e-6) - z * a

    ri = jax.lax.broadcasted_iota(jnp.int32, (STEPS, STEPS), 0)
    ci = jax.lax.broadcasted_iota(jnp.int32, (STEPS, STEPS), 1)
    tril = (ri >= ci).astype(jnp.float32)

    nzd = noise_ref[...] + drift_dt
    s = jax.lax.dot(tril, nzd,
                    precision=jax.lax.Precision.HIGHEST)

    t_f = jax.lax.broadcasted_iota(jnp.int32, (STEPS, BN), 0).astype(jnp.float32)
    enc_u = jnp.where(s >= th_hi, t_f, BIGF)
    enc_l = jnp.where(s <= th_lo, t_f, BIGF)
    mu = jnp.min(enc_u, axis=0, keepdims=True)
    ml = jnp.min(enc_l, axis=0, keepdims=True)

    t_first = jnp.minimum(mu, ml)
    hit = t_first < BIGF
    pr_ref[...] = jnp.where(hit, t_first * DT + ndt, MAX_T + ndt)
    pc_ref[...] = jnp.where(hit, jnp.where(mu <= ml, 1.0, 0.0), 0.5)


@jax.jit
def kernel(x, a, z, ndt, drift_gain, noise):
    n = x.shape[0]
    x2 = x.reshape(1, n)
    grid = (pl.cdiv(n, BN),)
    scal = pl.BlockSpec(memory_space=pltpu.SMEM)
    pr, pc = pl.pallas_call(
        _ddm_block,
        grid=grid,
        in_specs=[
            pl.BlockSpec((1, BN), lambda i: (0, i)),
            scal, scal, scal, scal,
            pl.BlockSpec((STEPS, BN), lambda i: (0, i)),
        ],
        out_specs=[
            pl.BlockSpec((1, BN), lambda i: (0, i)),
            pl.BlockSpec((1, BN), lambda i: (0, i)),
        ],
        out_shape=[
            jax.ShapeDtypeStruct((1, n), jnp.float32),
            jax.ShapeDtypeStruct((1, n), jnp.float32),
        ],
    )(x2,
      a.reshape(1, 1), z.reshape(1, 1), ndt.reshape(1, 1),
      drift_gain.reshape(1, 1), noise)
    return pr.reshape(n), pc.reshape(n)


# speedup vs baseline: 15.0314x; 1.1964x over previous
import jax
import jax.numpy as jnp
from jax.experimental import pallas as pl
from jax.experimental.pallas import tpu as pltpu

DT = 0.01
MAX_T = 2.0
STEPS = 200
BN = 2048
BIGF = 1e9


def _ddm_block(x_ref, a_ref, z_ref, ndt_ref, g_ref, noise_ref, pr_ref, pc_ref):
    a = a_ref[0, 0]
    z = z_ref[0, 0]
    ndt = ndt_ref[0, 0]
    gain = g_ref[0, 0]

    drift_dt = (gain * DT) * x_ref[...]
    th_hi = (a - 1e-6) - z * a
    th_lo = (-a + 1e-6) - z * a

    ri = jax.lax.broadcasted_iota(jnp.int32, (STEPS, STEPS), 0)
    ci = jax.lax.broadcasted_iota(jnp.int32, (STEPS, STEPS), 1)
    tril = (ri >= ci).astype(jnp.bfloat16)

    nzd = noise_ref[...] + drift_dt
    hi = nzd.astype(jnp.bfloat16)
    r1 = nzd - hi.astype(jnp.float32)
    mid = r1.astype(jnp.bfloat16)
    lo = (r1 - mid.astype(jnp.float32)).astype(jnp.bfloat16)

    def mm(b):
        return jax.lax.dot(tril, b, preferred_element_type=jnp.float32)

    s = mm(hi) + mm(mid) + mm(lo)

    t_f = jax.lax.broadcasted_iota(jnp.int32, (STEPS, BN), 0).astype(jnp.float32)
    enc_u = jnp.where(s >= th_hi, t_f, BIGF)
    enc_l = jnp.where(s <= th_lo, t_f, BIGF)
    mu = jnp.min(enc_u, axis=0, keepdims=True)
    ml = jnp.min(enc_l, axis=0, keepdims=True)

    t_first = jnp.minimum(mu, ml)
    hit = t_first < BIGF
    pr_ref[...] = jnp.where(hit, t_first * DT + ndt, MAX_T + ndt)
    pc_ref[...] = jnp.where(hit, jnp.where(mu <= ml, 1.0, 0.0), 0.5)


@jax.jit
def kernel(x, a, z, ndt, drift_gain, noise):
    n = x.shape[0]
    x2 = x.reshape(1, n)
    grid = (pl.cdiv(n, BN),)
    scal = pl.BlockSpec(memory_space=pltpu.SMEM)
    pr, pc = pl.pallas_call(
        _ddm_block,
        grid=grid,
        in_specs=[
            pl.BlockSpec((1, BN), lambda i: (0, i)),
            scal, scal, scal, scal,
            pl.BlockSpec((STEPS, BN), lambda i: (0, i)),
        ],
        out_specs=[
            pl.BlockSpec((1, BN), lambda i: (0, i)),
            pl.BlockSpec((1, BN), lambda i: (0, i)),
        ],
        out_shape=[
            jax.ShapeDtypeStruct((1, n), jnp.float32),
            jax.ShapeDtypeStruct((1, n), jnp.float32),
        ],
    )(x2,
      a.reshape(1, 1), z.reshape(1, 1), ndt.reshape(1, 1),
      drift_gain.reshape(1, 1), noise)
    return pr.reshape(n), pc.reshape(n)


# speedup vs baseline: 15.6921x; 1.0440x over previous
import jax
import jax.numpy as jnp
from jax.experimental import pallas as pl
from jax.experimental.pallas import tpu as pltpu

DT = 0.01
MAX_T = 2.0
STEPS = 200
BN = 4096
BIGF = 1e9


def _ddm_block(x_ref, a_ref, z_ref, ndt_ref, g_ref, noise_ref, pr_ref, pc_ref):
    a = a_ref[0, 0]
    z = z_ref[0, 0]
    ndt = ndt_ref[0, 0]
    gain = g_ref[0, 0]

    drift_dt = (gain * DT) * x_ref[...]
    th_hi = (a - 1e-6) - z * a
    th_lo = (-a + 1e-6) - z * a

    ri = jax.lax.broadcasted_iota(jnp.int32, (STEPS, STEPS), 0)
    ci = jax.lax.broadcasted_iota(jnp.int32, (STEPS, STEPS), 1)
    tril = (ri >= ci).astype(jnp.bfloat16)

    nzd = noise_ref[...] + drift_dt
    hi = nzd.astype(jnp.bfloat16)
    r1 = nzd - hi.astype(jnp.float32)
    mid = r1.astype(jnp.bfloat16)
    lo = (r1 - mid.astype(jnp.float32)).astype(jnp.bfloat16)

    def mm(b):
        return jax.lax.dot(tril, b, preferred_element_type=jnp.float32)

    s = mm(hi) + mm(mid) + mm(lo)

    t_f = jax.lax.broadcasted_iota(jnp.int32, (STEPS, BN), 0).astype(jnp.float32)
    enc_u = jnp.where(s >= th_hi, t_f, BIGF)
    enc_l = jnp.where(s <= th_lo, t_f, BIGF)
    mu = jnp.min(enc_u, axis=0, keepdims=True)
    ml = jnp.min(enc_l, axis=0, keepdims=True)

    t_first = jnp.minimum(mu, ml)
    hit = t_first < BIGF
    pr_ref[...] = jnp.where(hit, t_first * DT + ndt, MAX_T + ndt)
    pc_ref[...] = jnp.where(hit, jnp.where(mu <= ml, 1.0, 0.0), 0.5)


@jax.jit
def kernel(x, a, z, ndt, drift_gain, noise):
    n = x.shape[0]
    x2 = x.reshape(1, n)
    grid = (pl.cdiv(n, BN),)
    scal = pl.BlockSpec(memory_space=pltpu.SMEM)
    pr, pc = pl.pallas_call(
        _ddm_block,
        grid=grid,
        in_specs=[
            pl.BlockSpec((1, BN), lambda i: (0, i)),
            scal, scal, scal, scal,
            pl.BlockSpec((STEPS, BN), lambda i: (0, i)),
        ],
        out_specs=[
            pl.BlockSpec((1, BN), lambda i: (0, i)),
            pl.BlockSpec((1, BN), lambda i: (0, i)),
        ],
        out_shape=[
            jax.ShapeDtypeStruct((1, n), jnp.float32),
            jax.ShapeDtypeStruct((1, n), jnp.float32),
        ],
    )(x2,
      a.reshape(1, 1), z.reshape(1, 1), ndt.reshape(1, 1),
      drift_gain.reshape(1, 1), noise)
    return pr.reshape(n), pc.reshape(n)
